# Initial kernel scaffold; baseline (speedup 1.0000x reference)
import jax
import jax.numpy as jnp
from jax.experimental import pallas as pl


def kernel(attn_area, relative_position_bias_table, relative_position_index):
    raise NotImplementedError("write your pallas kernel here")


# speedup vs baseline: 11.9365x; 11.9365x over previous
import functools

import jax
import jax.numpy as jnp
from jax import lax
from jax.experimental import pallas as pl
from jax.experimental.pallas import tpu as pltpu
from jax.experimental.pallas import tpu_sc as plsc

WIN = 32
AREA = WIN * WIN
B = AREA * AREA
H = 16
TBL = (2 * WIN - 1) ** 2
TBL_W = TBL * H

NC, NS, L = 2, 16, 16
NW = NC * NS
PER_W = B // NW
C = 1024
NCH = PER_W // C


def _sc_body(tbl_hbm, idx_hbm, out_hbm, tbl_v, idx_v, out_v,
             tbl_sem, isem0, isem1, osem0, osem1):
    isems = (isem0, isem1)
    osems = (osem0, osem1)
    wid = lax.axis_index("s") * NC + lax.axis_index("c")
    base = wid * PER_W

    tbl_cp = pltpu.async_copy(tbl_hbm, tbl_v, tbl_sem)

    def fire_idx(s, b):
        pltpu.async_copy(idx_hbm.at[pl.ds(base + s * C, C)], idx_v.at[b],
                         isems[b])

    def wait_idx(b):
        pltpu.make_async_copy(idx_hbm.at[pl.ds(0, C)], idx_v.at[b],
                              isems[b]).wait()

    def fire_out(s, b):
        pltpu.async_copy(out_v.at[b], out_hbm.at[:, pl.ds(base + s * C, C)],
                         osems[b])

    def wait_out(b):
        pltpu.make_async_copy(out_hbm.at[:, pl.ds(0, C)], out_v.at[b],
                              osems[b]).wait()

    def compute(b):
        def vbody(v, carry):
            iv = idx_v[b, pl.ds(v * L, L)]
            a0 = iv * jnp.int32(H)
            for h in range(H):
                out_v[b, h, pl.ds(v * L, L)] = plsc.load_gather(
                    tbl_v, [a0 + jnp.int32(h)])
            return carry
        lax.fori_loop(0, C // L, vbody, jnp.int32(0))

    fire_idx(0, 0)
    fire_idx(1, 1)
    tbl_cp.wait()

    for b in (0, 1):
        wait_idx(b)
        compute(b)
        fire_out(b, b)
        fire_idx(b + 2, b)

    @pl.loop(2, NCH, step=2)
    def _chunks(s):
        for b in (0, 1):
            sb = s + b
            wait_idx(b)
            wait_out(b)
            compute(b)
            fire_out(sb, b)

            @pl.when(sb + 2 < NCH)
            def _():
                fire_idx(sb + 2, b)

    wait_out(0)
    wait_out(1)


@functools.cache
def _build():
    mesh = plsc.VectorSubcoreMesh(core_axis_name="c", subcore_axis_name="s")
    return pl.kernel(
        _sc_body,
        out_type=jax.ShapeDtypeStruct((H, B), jnp.float32),
        mesh=mesh,
        compiler_params=pltpu.CompilerParams(needs_layout_passes=False),
        scratch_types=[
            pltpu.VMEM((TBL_W,), jnp.float32),
            pltpu.VMEM((2, C), jnp.int32),
            pltpu.VMEM((2, H, C), jnp.float32),
            pltpu.SemaphoreType.DMA,
            pltpu.SemaphoreType.DMA,
            pltpu.SemaphoreType.DMA,
            pltpu.SemaphoreType.DMA,
            pltpu.SemaphoreType.DMA,
        ],
    )


def kernel(attn_area, relative_position_bias_table, relative_position_index):
    del attn_area
    tbl = relative_position_bias_table.reshape(TBL_W)
    idx = relative_position_index.reshape(B)
    out = _build()(tbl, idx)
    return out.reshape(1, H, AREA, AREA)


# speedup vs baseline: 22.3614x; 1.8734x over previous
import functools

import jax
import jax.numpy as jnp
from jax import lax
from jax.experimental import pallas as pl
from jax.experimental.pallas import tpu as pltpu
from jax.experimental.pallas import tpu_sc as plsc

WIN = 32
AREA = WIN * WIN
B = AREA * AREA
H = 16
TBL = (2 * WIN - 1) ** 2
TBL_W = TBL * H

NC, NS, L = 2, 16, 16
NW = NC * NS
PER_W = B // NW
C = 1024
NCH = PER_W // C


def _sc_body(tbl_hbm, idx_hbm, out_hbm, tbl_v, idx_v, out_v,
             tbl_sem, isem0, isem1, osem0, osem1):
    isems = (isem0, isem1)
    osems = (osem0, osem1)
    wid = lax.axis_index("s") * NC + lax.axis_index("c")
    base = wid * PER_W

    tbl_cp = pltpu.async_copy(tbl_hbm, tbl_v, tbl_sem)

    def fire_idx(s, b):
        pltpu.async_copy(idx_hbm.at[pl.ds(base + s * C, C)], idx_v.at[b],
                         isems[b])

    def wait_idx(b):
        pltpu.make_async_copy(idx_hbm.at[pl.ds(0, C)], idx_v.at[b],
                              isems[b]).wait()

    def fire_out(s, b):
        pltpu.async_copy(out_v.at[b], out_hbm.at[:, pl.ds(base + s * C, C)],
                         osems[b])

    def wait_out(b):
        pltpu.make_async_copy(out_hbm.at[:, pl.ds(0, C)], out_v.at[b],
                              osems[b]).wait()

    def compute(b):
        @plsc.parallel_loop(0, C // L, step=1, unroll=4)
        def vbody(v):
            iv = idx_v[b, pl.ds(v * L, L)]
            a0 = iv * jnp.int32(H)
            for h in range(H):
                out_v[b, h, pl.ds(v * L, L)] = plsc.load_gather(
                    tbl_v, [a0 + jnp.int32(h)])

    fire_idx(0, 0)
    fire_idx(1, 1)
    tbl_cp.wait()

    for b in (0, 1):
        wait_idx(b)
        compute(b)
        fire_out(b, b)
        fire_idx(b + 2, b)

    @pl.loop(2, NCH, step=2)
    def _chunks(s):
        for b in (0, 1):
            sb = s + b
            wait_idx(b)
            wait_out(b)
            compute(b)
            fire_out(sb, b)

            @pl.when(sb + 2 < NCH)
            def _():
                fire_idx(sb + 2, b)

    wait_out(0)
    wait_out(1)


@functools.cache
def _build():
    mesh = plsc.VectorSubcoreMesh(core_axis_name="c", subcore_axis_name="s")
    return pl.kernel(
        _sc_body,
        out_type=jax.ShapeDtypeStruct((H, B), jnp.float32),
        mesh=mesh,
        compiler_params=pltpu.CompilerParams(needs_layout_passes=False),
        scratch_types=[
            pltpu.VMEM((TBL_W,), jnp.float32),
            pltpu.VMEM((2, C), jnp.int32),
            pltpu.VMEM((2, H, C), jnp.float32),
            pltpu.SemaphoreType.DMA,
            pltpu.SemaphoreType.DMA,
            pltpu.SemaphoreType.DMA,
            pltpu.SemaphoreType.DMA,
            pltpu.SemaphoreType.DMA,
        ],
    )


def kernel(attn_area, relative_position_bias_table, relative_position_index):
    del attn_area
    tbl = relative_position_bias_table.reshape(TBL_W)
    idx = relative_position_index.reshape(B)
    out = _build()(tbl, idx)
    return out.reshape(1, H, AREA, AREA)


# speedup vs baseline: 31.9542x; 1.4290x over previous
import functools

import jax
import jax.numpy as jnp
from jax import lax
from jax.experimental import pallas as pl
from jax.experimental.pallas import tpu as pltpu
from jax.experimental.pallas import tpu_sc as plsc

WIN = 32
AREA = WIN * WIN
B = AREA * AREA
H = 16
TBL = (2 * WIN - 1) ** 2
TBL_W = TBL * H

NC, NS, L = 2, 16, 16
NW = NC * NS
PER_W = B // NW
C = 1024
NCH = PER_W // C


def _sc_body(tbl_hbm, idx_hbm, out_hbm, tbl_v, idx_v, out_v,
             tbl_sem, isem0, isem1, osem0, osem1):
    isems = (isem0, isem1)
    osems = (osem0, osem1)
    wid = lax.axis_index("s") * NC + lax.axis_index("c")
    base = wid * PER_W

    tbl_cp = pltpu.async_copy(tbl_hbm, tbl_v, tbl_sem)

    def fire_idx(s, b):
        pltpu.async_copy(idx_hbm.at[base // C + s], idx_v.at[b], isems[b])

    def wait_idx(b):
        pltpu.make_async_copy(idx_hbm.at[0], idx_v.at[b], isems[b]).wait()

    def fire_out(s, b):
        pltpu.async_copy(out_v.at[b], out_hbm.at[:, base // C + s, :],
                         osems[b])

    def wait_out(b):
        pltpu.make_async_copy(out_hbm.at[:, 0, :], out_v.at[b],
                              osems[b]).wait()

    def compute(b):
        @plsc.parallel_loop(0, C // L, step=1, unroll=4)
        def vbody(v):
            iv = idx_v[b, pl.ds(v * L, L)]
            a0 = iv * jnp.int32(H)
            for h in range(H):
                out_v[b, h, pl.ds(v * L, L)] = plsc.load_gather(
                    tbl_v, [a0 + jnp.int32(h)])

    fire_idx(0, 0)
    fire_idx(1, 1)
    tbl_cp.wait()

    for b in (0, 1):
        wait_idx(b)
        compute(b)
        fire_out(b, b)
        fire_idx(b + 2, b)

    @pl.loop(2, NCH, step=2)
    def _chunks(s):
        for b in (0, 1):
            sb = s + b
            wait_idx(b)
            wait_out(b)
            compute(b)
            fire_out(sb, b)

            @pl.when(sb + 2 < NCH)
            def _():
                fire_idx(sb + 2, b)

    wait_out(0)
    wait_out(1)


@functools.cache
def _build():
    mesh = plsc.VectorSubcoreMesh(core_axis_name="c", subcore_axis_name="s")
    return pl.kernel(
        _sc_body,
        out_type=jax.ShapeDtypeStruct((H, AREA, AREA), jnp.float32),
        mesh=mesh,
        compiler_params=pltpu.CompilerParams(needs_layout_passes=False),
        scratch_types=[
            pltpu.VMEM((TBL_W,), jnp.float32),
            pltpu.VMEM((2, C), jnp.int32),
            pltpu.VMEM((2, H, C), jnp.float32),
            pltpu.SemaphoreType.DMA,
            pltpu.SemaphoreType.DMA,
            pltpu.SemaphoreType.DMA,
            pltpu.SemaphoreType.DMA,
            pltpu.SemaphoreType.DMA,
        ],
    )


def kernel(attn_area, relative_position_bias_table, relative_position_index):
    del attn_area
    tbl = relative_position_bias_table.reshape(TBL_W)
    out = _build()(tbl, relative_position_index)
    return out[None]


# speedup vs baseline: 66.7441x; 2.0887x over previous
import functools

import jax
import jax.numpy as jnp
from jax import lax
from jax.experimental import pallas as pl
from jax.experimental.pallas import tpu as pltpu
from jax.experimental.pallas import tpu_sc as plsc

WIN = 32
AREA = WIN * WIN
B = AREA * AREA
H = 16
TBL = (2 * WIN - 1) ** 2
TBL_W = TBL * H

NC, NS, L = 2, 16, 16
NW = NC * NS
PER_W = B // NW
C = 1024
NCH = PER_W // C


def _sc_body(tbl_hbm, idx_hbm, out_hbm, tbl_v, idx_v, out_v,
             tbl_sem, isem0, isem1, osem0, osem1):
    isems = (isem0, isem1)
    osems = (osem0, osem1)
    wid = lax.axis_index("s") * NC + lax.axis_index("c")
    base = wid * PER_W

    tbl_cp = pltpu.async_copy(tbl_hbm, tbl_v, tbl_sem)

    def fire_idx(s, b):
        pltpu.async_copy(idx_hbm.at[base // C + s], idx_v.at[b], isems[b])

    def wait_idx(b):
        pltpu.make_async_copy(idx_hbm.at[0], idx_v.at[b], isems[b]).wait()

    def fire_out(s, b):
        pltpu.async_copy(out_v.at[b], out_hbm.at[:, base // C + s, :],
                         osems[b])

    def wait_out(b):
        pltpu.make_async_copy(out_hbm.at[:, 0, :], out_v.at[b],
                              osems[b]).wait()

    def compute(b):
        @plsc.parallel_loop(0, C // L, step=1, unroll=4)
        def vbody(v):
            iv = idx_v[b, pl.ds(v * L, L)]
            for h in range(H):
                out_v[b, h, pl.ds(v * L, L)] = plsc.load_gather(
                    tbl_v, [iv + jnp.int32(h * TBL)])

    fire_idx(0, 0)
    fire_idx(1, 1)
    tbl_cp.wait()

    for b in (0, 1):
        wait_idx(b)
        compute(b)
        fire_out(b, b)
        fire_idx(b + 2, b)

    @pl.loop(2, NCH, step=2)
    def _chunks(s):
        for b in (0, 1):
            sb = s + b
            wait_idx(b)
            wait_out(b)
            compute(b)
            fire_out(sb, b)

            @pl.when(sb + 2 < NCH)
            def _():
                fire_idx(sb + 2, b)

    wait_out(0)
    wait_out(1)


@functools.cache
def _build():
    mesh = plsc.VectorSubcoreMesh(core_axis_name="c", subcore_axis_name="s")
    return pl.kernel(
        _sc_body,
        out_type=jax.ShapeDtypeStruct((H, AREA, AREA), jnp.float32),
        mesh=mesh,
        compiler_params=pltpu.CompilerParams(needs_layout_passes=False),
        scratch_types=[
            pltpu.VMEM((TBL_W,), jnp.float32),
            pltpu.VMEM((2, C), jnp.int32),
            pltpu.VMEM((2, H, C), jnp.float32),
            pltpu.SemaphoreType.DMA,
            pltpu.SemaphoreType.DMA,
            pltpu.SemaphoreType.DMA,
            pltpu.SemaphoreType.DMA,
            pltpu.SemaphoreType.DMA,
        ],
    )


def kernel(attn_area, relative_position_bias_table, relative_position_index):
    del attn_area
    tbl = relative_position_bias_table.T.reshape(TBL_W)
    out = _build()(tbl, relative_position_index)
    return out[None]
